# baseline (speedup 1.0000x reference)
import jax
import jax.numpy as jnp
from jax import lax
from jax.experimental import pallas as pl
from jax.experimental.pallas import tpu as pltpu
from jax.experimental.pallas import tpu_sc as plsc

N = 50000
E = 800000
FEA = 128
HD = 32
H_DIM = 128
NS = 16
NPAD = 51200
STRIPE = NPAD // NS
CHUNK = 128
CPS = 392
IB = 14
NBLK = CPS // IB
NCHP = CPS * NS
E_PAD = NCHP * CHUNK
BLK = 400
GRID = N // BLK


def _sc_agg_build(with_cnt):
    mesh = plsc.VectorSubcoreMesh(core_axis_name="c", subcore_axis_name="s")
    outs = [jax.ShapeDtypeStruct((NPAD, HD), jnp.bfloat16),
            jax.ShapeDtypeStruct((NPAD, HD), jnp.bfloat16)]
    scratch = [
        pltpu.VMEM((2, IB, CHUNK), jnp.int32),
        pltpu.VMEM((2, IB, CHUNK), jnp.int32),
        pltpu.VMEM((CHUNK, HD), jnp.bfloat16),
        pltpu.VMEM((CHUNK, HD), jnp.bfloat16),
        pltpu.VMEM_SHARED((NPAD, HD), jnp.bfloat16),
        pltpu.SemaphoreType.DMA,
        pltpu.SemaphoreType.DMA,
        pltpu.SemaphoreType.DMA,
    ]
    if with_cnt:
        outs.append(jax.ShapeDtypeStruct((NPAD,), jnp.float32))
        scratch += [
            pltpu.VMEM((CHUNK,), jnp.float32),
            pltpu.VMEM((CHUNK,), jnp.float32),
            pltpu.VMEM_SHARED((NPAD,), jnp.float32),
        ]

    def body(hA, hB, src_h, dst_h, *rest):
        if with_cnt:
            (outA, outB, outC, src3, dst3, rows0, rows1, agg_sh,
             sem0, sem1, isem, cbuf, obuf, cnt_sh) = rest
        else:
            (outA, outB, src3, dst3, rows0, rows1, agg_sh,
             sem0, sem1, isem) = rest
        rows = (rows0, rows1)
        gsem = (sem0, sem1)
        cid = lax.axis_index("c")
        sid = lax.axis_index("s")
        z16 = jnp.zeros((16,), jnp.float32)
        zb32 = jnp.zeros((32,), jnp.bfloat16)

        def fill_row(i, c):
            rows0[i, pl.ds(0, 32)] = zb32
            return c
        lax.fori_loop(0, CHUNK, fill_row, 0)

        def zero_stripe(k, c):
            pltpu.sync_copy(rows0, agg_sh.at[pl.ds(sid * STRIPE + k * CHUNK, CHUNK)])
            return c
        lax.fori_loop(0, STRIPE // CHUNK, zero_stripe, 0)

        if with_cnt:
            o16 = jnp.ones((16,), jnp.float32)

            def fill_c(k, c):
                cbuf[pl.ds(k * 16, 16)] = z16
                obuf[pl.ds(k * 16, 16)] = o16
                return c
            lax.fori_loop(0, CHUNK // 16, fill_c, 0)

            @pl.when(cid == 0)
            def _():
                def zc(k, c):
                    pltpu.sync_copy(cbuf, cnt_sh.at[pl.ds(sid * STRIPE + k * CHUNK, CHUNK)])
                    return c
                lax.fori_loop(0, STRIPE // CHUNK, zc, 0)

        plsc.subcore_barrier()

        def main_loop(tab, do_cnt):
            def scat(sl, k):
                pltpu.sync_copy(rows[k % 2], agg_sh.at[dst3.at[sl, k]],
                                add=True)
                if do_cnt:
                    pltpu.sync_copy(obuf, cnt_sh.at[dst3.at[sl, k]],
                                    add=True)

            def block(sl, nsl, noff, valid_next):
                for k in range(IB):
                    par = k % 2
                    pltpu.make_async_copy(tab.at[src3.at[sl, k]],
                                          rows[par], gsem[par]).wait()
                    if k < IB - 1:
                        pltpu.async_copy(tab.at[src3.at[sl, k + 1]],
                                         rows[1 - par], gsem[1 - par])
                        scat(sl, k)
                    else:
                        def cross():
                            pltpu.make_async_copy(
                                src_h.at[pl.ds(noff, IB)], src3.at[nsl],
                                isem).wait()
                            pltpu.make_async_copy(
                                dst_h.at[pl.ds(noff, IB)], dst3.at[nsl],
                                isem).wait()
                            pltpu.async_copy(tab.at[src3.at[nsl, 0]],
                                             rows[1 - par], gsem[1 - par])
                        if valid_next is None:
                            cross()
                        else:
                            pl.when(valid_next)(cross)
                        scat(sl, k)

            c0 = sid * CPS
            pltpu.sync_copy(src_h.at[pl.ds(c0, IB)], src3.at[0])
            pltpu.sync_copy(dst_h.at[pl.ds(c0, IB)], dst3.at[0])
            pltpu.async_copy(tab.at[src3.at[0, 0]], rows[0], gsem[0])

            def dbl(T, c):
                base = c0 + 2 * T * IB
                pltpu.async_copy(src_h.at[pl.ds(base + IB, IB)],
                                 src3.at[1], isem)
                pltpu.async_copy(dst_h.at[pl.ds(base + IB, IB)],
                                 dst3.at[1], isem)
                block(0, 1, base + IB, None)
                last = T + 1 < NBLK // 2

                @pl.when(last)
                def _():
                    pltpu.async_copy(src_h.at[pl.ds(base + 2 * IB, IB)],
                                     src3.at[0], isem)
                    pltpu.async_copy(dst_h.at[pl.ds(base + 2 * IB, IB)],
                                     dst3.at[0], isem)
                block(1, 0, base + 2 * IB, last)
                return c
            lax.fori_loop(0, NBLK // 2, dbl, 0)

        @pl.when(cid == 0)
        def _():
            main_loop(hA, with_cnt)

        @pl.when(cid == 1)
        def _():
            main_loop(hB, False)

        plsc.subcore_barrier()

        @pl.when(cid == 0)
        def _():
            pltpu.sync_copy(agg_sh.at[pl.ds(sid * STRIPE, STRIPE)],
                            outA.at[pl.ds(sid * STRIPE, STRIPE)])
            if with_cnt:
                pltpu.sync_copy(cnt_sh.at[pl.ds(sid * STRIPE, STRIPE)],
                                outC.at[pl.ds(sid * STRIPE, STRIPE)])

        @pl.when(cid == 1)
        def _():
            pltpu.sync_copy(agg_sh.at[pl.ds(sid * STRIPE, STRIPE)],
                            outB.at[pl.ds(sid * STRIPE, STRIPE)])

    return pl.kernel(body, out_type=tuple(outs), mesh=mesh,
                     scratch_types=tuple(scratch),
                     compiler_params=pltpu.CompilerParams(
                         use_tc_tiling_on_sc=False))


_SC_AGG_CNT = _sc_agg_build(True)
_SC_AGG = _sc_agg_build(False)


NP4 = N // 4
NPAD4 = NPAD // 4
BLK4 = 512
GRID4 = NPAD4 // BLK4


def _emb(xp, wAb, wBb, bA4, bB4):
    def body(x_r, wA_r, wB_r, bA_r, bB_r, oA_r, oB_r):
        xb = jnp.nan_to_num(x_r[...])
        oA_r[...] = jnp.dot(xb, wA_r[...],
                            preferred_element_type=jnp.float32, precision=lax.Precision.HIGHEST) + bA_r[...]
        oB_r[...] = jnp.dot(xb, wB_r[...],
                            preferred_element_type=jnp.float32, precision=lax.Precision.HIGHEST) + bB_r[...]

    blk = lambda i: (i, 0)
    one = lambda i: (0, 0)
    return pl.pallas_call(
        body,
        grid=(GRID4,),
        in_specs=[pl.BlockSpec((BLK4, 4 * FEA), blk),
                  pl.BlockSpec((4 * FEA, 4 * HD), one),
                  pl.BlockSpec((4 * FEA, 4 * HD), one),
                  pl.BlockSpec((1, 4 * HD), one),
                  pl.BlockSpec((1, 4 * HD), one)],
        out_specs=[pl.BlockSpec((BLK4, 4 * HD), blk)] * 2,
        out_shape=[jax.ShapeDtypeStruct((NPAD4, 4 * HD), jnp.float32)] * 2,
    )(xp, wAb, wBb, bA4, bB4)


def _update_build(with_sum):
    def body(aggA, aggB, cnt4, rsel, hA, hB, w00, w01, w10, w11,
             r00, r01, r10, r11, bA, bB, oA, oB, *s):
        i = pl.program_id(0)
        f32 = jnp.float32
        inv = jnp.dot(1.0 / jnp.maximum(cnt4[...], 1.0), rsel[...],
                      preferred_element_type=f32, precision=lax.Precision.HIGHEST)
        mA = aggA[...].astype(f32) * inv
        mB = aggB[...].astype(f32) * inv
        ha = hA[...]
        hb = hB[...]
        nA = (jnp.dot(mA, w00[...], preferred_element_type=f32, precision=lax.Precision.HIGHEST)
              + jnp.dot(mB, w10[...], preferred_element_type=f32, precision=lax.Precision.HIGHEST)
              + jnp.dot(ha, r00[...], preferred_element_type=f32, precision=lax.Precision.HIGHEST)
              + jnp.dot(hb, r10[...], preferred_element_type=f32, precision=lax.Precision.HIGHEST)
              + bA[...])
        nB = (jnp.dot(mA, w01[...], preferred_element_type=f32, precision=lax.Precision.HIGHEST)
              + jnp.dot(mB, w11[...], preferred_element_type=f32, precision=lax.Precision.HIGHEST)
              + jnp.dot(ha, r01[...], preferred_element_type=f32, precision=lax.Precision.HIGHEST)
              + jnp.dot(hb, r11[...], preferred_element_type=f32, precision=lax.Precision.HIGHEST)
              + bB[...])
        oA[...] = nA
        oB[...] = nB
        if with_sum:
            sA, sB = s

            @pl.when(i == 0)
            def _():
                sA[...] = jnp.zeros_like(sA)
                sB[...] = jnp.zeros_like(sB)

            live = (lax.broadcasted_iota(jnp.int32, (BLK4, 1), 0)
                    + i * BLK4) < NP4
            sA[...] += jnp.sum(jnp.where(live, nA, 0.0), axis=0,
                               keepdims=True)
            sB[...] += jnp.sum(jnp.where(live, nB, 0.0), axis=0,
                               keepdims=True)

    blk = lambda i: (i, 0)
    one = lambda i: (0, 0)
    in_specs = ([pl.BlockSpec((BLK4, 4 * HD), blk)] * 2
                + [pl.BlockSpec((BLK4, 4), blk),
                   pl.BlockSpec((4, 4 * HD), one)]
                + [pl.BlockSpec((BLK4, 4 * HD), blk)] * 2
                + [pl.BlockSpec((4 * HD, 4 * HD), one)] * 8
                + [pl.BlockSpec((1, 4 * HD), one)] * 2)
    out_specs = [pl.BlockSpec((BLK4, 4 * HD), blk)] * 2
    out_shape = [jax.ShapeDtypeStruct((NPAD4, 4 * HD), jnp.float32)] * 2
    if with_sum:
        out_specs += [pl.BlockSpec((1, 4 * HD), one)] * 2
        out_shape += [jax.ShapeDtypeStruct((1, 4 * HD), jnp.float32)] * 2

    return pl.pallas_call(body, grid=(GRID4,), in_specs=in_specs,
                          out_specs=out_specs, out_shape=out_shape)


_UPDATE = _update_build(False)
_UPDATE_SUM = _update_build(True)


def _final(sA, sB, wfA, wfB, bfc, wout, bout):
    def body(sA_r, sB_r, wfA_r, wfB_r, bfc_r, wo_r, bo_r, o_r):
        f32 = jnp.float32
        scale = 1.0 / N
        g = (jnp.dot(sA_r[...] * scale, wfA_r[...], preferred_element_type=f32, precision=lax.Precision.HIGHEST)
             + jnp.dot(sB_r[...] * scale, wfB_r[...], preferred_element_type=f32, precision=lax.Precision.HIGHEST)
             + bfc_r[...])
        sp = jnp.maximum(g, 0.0) + jnp.log(1.0 + jnp.exp(-jnp.abs(g)))
        o_r[...] = jnp.dot(sp, wo_r[...], preferred_element_type=f32, precision=lax.Precision.HIGHEST) + bo_r[...]

    return pl.pallas_call(
        body, out_shape=jax.ShapeDtypeStruct((1, 1), jnp.float32),
    )(sA, sB, wfA, wfB, bfc, wout, bout)


def _blockdiag(w):
    return jnp.kron(jnp.eye(4, dtype=w.dtype), w)


def kernel(x, edge_index, W_emb, b_emb, Wl0, bl0, Wr0, Wl1, bl1, Wr1,
           Wl2, bl2, Wr2, W_fc, b_fc, W_out, b_out):
    src = jnp.concatenate(
        [edge_index[0].astype(jnp.int32),
         jnp.zeros((E_PAD - E,), jnp.int32)]).reshape(NCHP, CHUNK)
    dst = jnp.concatenate(
        [edge_index[1].astype(jnp.int32),
         jnp.full((E_PAD - E,), N, jnp.int32)]).reshape(NCHP, CHUNK)

    eye4 = jnp.eye(4, dtype=jnp.float32)
    wAb = jnp.kron(eye4, W_emb[:, :HD])
    wBb = jnp.kron(eye4, W_emb[:, HD:])
    bA4 = jnp.tile(b_emb[:HD], 4).reshape(1, 4 * HD)
    bB4 = jnp.tile(b_emb[HD:], 4).reshape(1, 4 * HD)
    rsel = jnp.kron(eye4, jnp.ones((1, HD), jnp.float32))
    ssel = jnp.kron(jnp.ones((4, 1), jnp.float32),
                    jnp.eye(HD, dtype=jnp.float32))
    wfA = ssel @ W_fc[:HD]
    wfB = ssel @ W_fc[HD:]

    xp = jnp.pad(x.reshape(NP4, 4 * FEA), ((0, NPAD4 - NP4), (0, 0)))
    hA, hB = _emb(xp, wAb, wBb, bA4, bB4)

    cnt4 = None
    sA = sB = None
    for li, (Wl, bl, Wr) in enumerate(((Wl0, bl0, Wr0), (Wl1, bl1, Wr1),
                                       (Wl2, bl2, Wr2))):
        hAl = hA.reshape(NPAD, HD).astype(jnp.bfloat16)
        hBl = hB.reshape(NPAD, HD).astype(jnp.bfloat16)
        if li == 0:
            aggA, aggB, cnt = _SC_AGG_CNT(hAl, hBl, src, dst)
            cnt4 = cnt.reshape(NPAD4, 4)
        else:
            aggA, aggB = _SC_AGG(hAl, hBl, src, dst)
        args = (aggA.reshape(NPAD4, 4 * HD), aggB.reshape(NPAD4, 4 * HD),
                cnt4, rsel, hA, hB,
                _blockdiag(Wl[:HD, :HD]), _blockdiag(Wl[:HD, HD:]),
                _blockdiag(Wl[HD:, :HD]), _blockdiag(Wl[HD:, HD:]),
                _blockdiag(Wr[:HD, :HD]), _blockdiag(Wr[:HD, HD:]),
                _blockdiag(Wr[HD:, :HD]), _blockdiag(Wr[HD:, HD:]),
                jnp.tile(bl[:HD], 4).reshape(1, 4 * HD),
                jnp.tile(bl[HD:], 4).reshape(1, 4 * HD))
        if li < 2:
            hA, hB = _UPDATE(*args)
        else:
            hA, hB, sA, sB = _UPDATE_SUM(*args)
    out = _final(sA, sB, wfA, wfB, b_fc.reshape(1, H_DIM),
                 W_out, b_out.reshape(1, 1))
    return out.reshape(1)


# speedup vs baseline: 1.2546x; 1.2546x over previous
import jax
import jax.numpy as jnp
from jax import lax
from jax.experimental import pallas as pl
from jax.experimental.pallas import tpu as pltpu
from jax.experimental.pallas import tpu_sc as plsc

N = 50000
E = 800000
FEA = 128
HD = 32
H_DIM = 128
NS = 16
NPAD = 51200
STRIPE = NPAD // NS
CHUNK = 128
CPS = 396
IB = 12
NBLK = CPS // IB
NCHP = CPS * NS
E_PAD = NCHP * CHUNK
BLK = 400
GRID = N // BLK


def _sc_agg_build(with_cnt):
    mesh = plsc.VectorSubcoreMesh(core_axis_name="c", subcore_axis_name="s")
    outs = [jax.ShapeDtypeStruct((NPAD, HD), jnp.bfloat16),
            jax.ShapeDtypeStruct((NPAD, HD), jnp.bfloat16)]
    scratch = [
        pltpu.VMEM((3, IB, CHUNK), jnp.int32),
        pltpu.VMEM((3, IB, CHUNK), jnp.int32),
        pltpu.VMEM((CHUNK, HD), jnp.bfloat16),
        pltpu.VMEM((CHUNK, HD), jnp.bfloat16),
        pltpu.VMEM((CHUNK, HD), jnp.bfloat16),
        pltpu.VMEM_SHARED((NPAD, HD), jnp.bfloat16),
        pltpu.SemaphoreType.DMA,
        pltpu.SemaphoreType.DMA,
        pltpu.SemaphoreType.DMA,
        pltpu.SemaphoreType.DMA,
    ]
    if with_cnt:
        outs.append(jax.ShapeDtypeStruct((NPAD,), jnp.float32))
        scratch += [
            pltpu.VMEM((CHUNK,), jnp.float32),
            pltpu.VMEM((CHUNK,), jnp.float32),
            pltpu.VMEM_SHARED((NPAD,), jnp.float32),
        ]

    def body(hA, hB, src_h, dst_h, *rest):
        if with_cnt:
            (outA, outB, outC, src3, dst3, rows0, rows1, rows2, agg_sh,
             sem0, sem1, sem2, isem, cbuf, obuf, cnt_sh) = rest
        else:
            (outA, outB, src3, dst3, rows0, rows1, rows2, agg_sh,
             sem0, sem1, sem2, isem) = rest
        rows = (rows0, rows1, rows2)
        gsem = (sem0, sem1, sem2)
        cid = lax.axis_index("c")
        sid = lax.axis_index("s")
        z16 = jnp.zeros((16,), jnp.float32)
        zb32 = jnp.zeros((32,), jnp.bfloat16)

        def fill_row(i, c):
            rows0[i, pl.ds(0, 32)] = zb32
            return c
        lax.fori_loop(0, CHUNK, fill_row, 0)

        def zero_stripe(k, c):
            pltpu.sync_copy(rows0, agg_sh.at[pl.ds(sid * STRIPE + k * CHUNK, CHUNK)])
            return c
        lax.fori_loop(0, STRIPE // CHUNK, zero_stripe, 0)

        if with_cnt:
            o16 = jnp.ones((16,), jnp.float32)

            def fill_c(k, c):
                cbuf[pl.ds(k * 16, 16)] = z16
                obuf[pl.ds(k * 16, 16)] = o16
                return c
            lax.fori_loop(0, CHUNK // 16, fill_c, 0)

            @pl.when(cid == 0)
            def _():
                def zc(k, c):
                    pltpu.sync_copy(cbuf, cnt_sh.at[pl.ds(sid * STRIPE + k * CHUNK, CHUNK)])
                    return c
                lax.fori_loop(0, STRIPE // CHUNK, zc, 0)

        plsc.subcore_barrier()

        def main_loop(tab, do_cnt):
            def block(sl, nsl, noff, valid_next):
                for k in range(IB):
                    q = k % 3
                    pltpu.make_async_copy(tab.at[src3.at[sl, k]],
                                          rows[q], gsem[q]).wait()
                    nq = (k + 2) % 3
                    if k == IB - 3:
                        def wait_idx():
                            pltpu.make_async_copy(
                                src_h.at[pl.ds(noff, IB)], src3.at[nsl],
                                isem).wait()
                            pltpu.make_async_copy(
                                dst_h.at[pl.ds(noff, IB)], dst3.at[nsl],
                                isem).wait()
                        if valid_next is None:
                            wait_idx()
                        else:
                            pl.when(valid_next)(wait_idx)
                    if k < IB - 2:
                        pltpu.async_copy(tab.at[src3.at[sl, k + 2]],
                                         rows[nq], gsem[nq])
                    else:
                        def cross():
                            pltpu.async_copy(
                                tab.at[src3.at[nsl, k + 2 - IB]],
                                rows[nq], gsem[nq])
                        if valid_next is None:
                            cross()
                        else:
                            pl.when(valid_next)(cross)
                    pltpu.sync_copy(rows[q], agg_sh.at[dst3.at[sl, k]],
                                    add=True)
                    if do_cnt:
                        pltpu.sync_copy(obuf, cnt_sh.at[dst3.at[sl, k]],
                                        add=True)

            c0 = sid * CPS
            pltpu.sync_copy(src_h.at[pl.ds(c0, IB)], src3.at[0])
            pltpu.sync_copy(dst_h.at[pl.ds(c0, IB)], dst3.at[0])
            pltpu.async_copy(tab.at[src3.at[0, 0]], rows[0], gsem[0])
            pltpu.async_copy(tab.at[src3.at[0, 1]], rows[1], gsem[1])

            def trpl(T, c):
                base = c0 + 3 * T * IB
                pltpu.async_copy(src_h.at[pl.ds(base + IB, IB)],
                                 src3.at[1], isem)
                pltpu.async_copy(dst_h.at[pl.ds(base + IB, IB)],
                                 dst3.at[1], isem)
                block(0, 1, base + IB, None)
                pltpu.async_copy(src_h.at[pl.ds(base + 2 * IB, IB)],
                                 src3.at[2], isem)
                pltpu.async_copy(dst_h.at[pl.ds(base + 2 * IB, IB)],
                                 dst3.at[2], isem)
                block(1, 2, base + 2 * IB, None)
                nxt = T + 1 < NBLK // 3

                @pl.when(nxt)
                def _():
                    pltpu.async_copy(src_h.at[pl.ds(base + 3 * IB, IB)],
                                     src3.at[0], isem)
                    pltpu.async_copy(dst_h.at[pl.ds(base + 3 * IB, IB)],
                                     dst3.at[0], isem)
                block(2, 0, base + 3 * IB, nxt)
                return c
            lax.fori_loop(0, NBLK // 3, trpl, 0)

        @pl.when(cid == 0)
        def _():
            main_loop(hA, with_cnt)

        @pl.when(cid == 1)
        def _():
            main_loop(hB, False)

        plsc.subcore_barrier()

        @pl.when(cid == 0)
        def _():
            pltpu.sync_copy(agg_sh.at[pl.ds(sid * STRIPE, STRIPE)],
                            outA.at[pl.ds(sid * STRIPE, STRIPE)])
            if with_cnt:
                pltpu.sync_copy(cnt_sh.at[pl.ds(sid * STRIPE, STRIPE)],
                                outC.at[pl.ds(sid * STRIPE, STRIPE)])

        @pl.when(cid == 1)
        def _():
            pltpu.sync_copy(agg_sh.at[pl.ds(sid * STRIPE, STRIPE)],
                            outB.at[pl.ds(sid * STRIPE, STRIPE)])

    return pl.kernel(body, out_type=tuple(outs), mesh=mesh,
                     scratch_types=tuple(scratch),
                     compiler_params=pltpu.CompilerParams(
                         use_tc_tiling_on_sc=False))


_SC_AGG_CNT = _sc_agg_build(True)
_SC_AGG = _sc_agg_build(False)


NP4 = N // 4
NPAD4 = NPAD // 4
BLK4 = 512
GRID4 = NPAD4 // BLK4


def _emb(xp, wAb, wBb, bA4, bB4):
    def body(x_r, wA_r, wB_r, bA_r, bB_r, oA_r, oB_r):
        xb = jnp.nan_to_num(x_r[...])
        oA_r[...] = jnp.dot(xb, wA_r[...],
                            preferred_element_type=jnp.float32, precision=lax.Precision.HIGHEST) + bA_r[...]
        oB_r[...] = jnp.dot(xb, wB_r[...],
                            preferred_element_type=jnp.float32, precision=lax.Precision.HIGHEST) + bB_r[...]

    blk = lambda i: (i, 0)
    one = lambda i: (0, 0)
    return pl.pallas_call(
        body,
        grid=(GRID4,),
        in_specs=[pl.BlockSpec((BLK4, 4 * FEA), blk),
                  pl.BlockSpec((4 * FEA, 4 * HD), one),
                  pl.BlockSpec((4 * FEA, 4 * HD), one),
                  pl.BlockSpec((1, 4 * HD), one),
                  pl.BlockSpec((1, 4 * HD), one)],
        out_specs=[pl.BlockSpec((BLK4, 4 * HD), blk)] * 2,
        out_shape=[jax.ShapeDtypeStruct((NPAD4, 4 * HD), jnp.float32)] * 2,
    )(xp, wAb, wBb, bA4, bB4)


def _update_build(with_sum):
    def body(aggA, aggB, cnt4, rsel, hA, hB, w00, w01, w10, w11,
             r00, r01, r10, r11, bA, bB, oA, oB, *s):
        i = pl.program_id(0)
        f32 = jnp.float32
        inv = jnp.dot(1.0 / jnp.maximum(cnt4[...], 1.0), rsel[...],
                      preferred_element_type=f32, precision=lax.Precision.HIGHEST)
        mA = aggA[...].astype(f32) * inv
        mB = aggB[...].astype(f32) * inv
        ha = hA[...]
        hb = hB[...]
        nA = (jnp.dot(mA, w00[...], preferred_element_type=f32, precision=lax.Precision.HIGHEST)
              + jnp.dot(mB, w10[...], preferred_element_type=f32, precision=lax.Precision.HIGHEST)
              + jnp.dot(ha, r00[...], preferred_element_type=f32, precision=lax.Precision.HIGHEST)
              + jnp.dot(hb, r10[...], preferred_element_type=f32, precision=lax.Precision.HIGHEST)
              + bA[...])
        nB = (jnp.dot(mA, w01[...], preferred_element_type=f32, precision=lax.Precision.HIGHEST)
              + jnp.dot(mB, w11[...], preferred_element_type=f32, precision=lax.Precision.HIGHEST)
              + jnp.dot(ha, r01[...], preferred_element_type=f32, precision=lax.Precision.HIGHEST)
              + jnp.dot(hb, r11[...], preferred_element_type=f32, precision=lax.Precision.HIGHEST)
              + bB[...])
        oA[...] = nA
        oB[...] = nB
        if with_sum:
            sA, sB = s

            @pl.when(i == 0)
            def _():
                sA[...] = jnp.zeros_like(sA)
                sB[...] = jnp.zeros_like(sB)

            live = (lax.broadcasted_iota(jnp.int32, (BLK4, 1), 0)
                    + i * BLK4) < NP4
            sA[...] += jnp.sum(jnp.where(live, nA, 0.0), axis=0,
                               keepdims=True)
            sB[...] += jnp.sum(jnp.where(live, nB, 0.0), axis=0,
                               keepdims=True)

    blk = lambda i: (i, 0)
    one = lambda i: (0, 0)
    in_specs = ([pl.BlockSpec((BLK4, 4 * HD), blk)] * 2
                + [pl.BlockSpec((BLK4, 4), blk),
                   pl.BlockSpec((4, 4 * HD), one)]
                + [pl.BlockSpec((BLK4, 4 * HD), blk)] * 2
                + [pl.BlockSpec((4 * HD, 4 * HD), one)] * 8
                + [pl.BlockSpec((1, 4 * HD), one)] * 2)
    out_specs = [pl.BlockSpec((BLK4, 4 * HD), blk)] * 2
    out_shape = [jax.ShapeDtypeStruct((NPAD4, 4 * HD), jnp.float32)] * 2
    if with_sum:
        out_specs += [pl.BlockSpec((1, 4 * HD), one)] * 2
        out_shape += [jax.ShapeDtypeStruct((1, 4 * HD), jnp.float32)] * 2

    return pl.pallas_call(body, grid=(GRID4,), in_specs=in_specs,
                          out_specs=out_specs, out_shape=out_shape)


_UPDATE = _update_build(False)
_UPDATE_SUM = _update_build(True)


def _final(sA, sB, wfA, wfB, bfc, wout, bout):
    def body(sA_r, sB_r, wfA_r, wfB_r, bfc_r, wo_r, bo_r, o_r):
        f32 = jnp.float32
        scale = 1.0 / N
        g = (jnp.dot(sA_r[...] * scale, wfA_r[...], preferred_element_type=f32, precision=lax.Precision.HIGHEST)
             + jnp.dot(sB_r[...] * scale, wfB_r[...], preferred_element_type=f32, precision=lax.Precision.HIGHEST)
             + bfc_r[...])
        sp = jnp.maximum(g, 0.0) + jnp.log(1.0 + jnp.exp(-jnp.abs(g)))
        o_r[...] = jnp.dot(sp, wo_r[...], preferred_element_type=f32, precision=lax.Precision.HIGHEST) + bo_r[...]

    return pl.pallas_call(
        body, out_shape=jax.ShapeDtypeStruct((1, 1), jnp.float32),
    )(sA, sB, wfA, wfB, bfc, wout, bout)


def _blockdiag(w):
    return jnp.kron(jnp.eye(4, dtype=w.dtype), w)


def kernel(x, edge_index, W_emb, b_emb, Wl0, bl0, Wr0, Wl1, bl1, Wr1,
           Wl2, bl2, Wr2, W_fc, b_fc, W_out, b_out):
    src = jnp.concatenate(
        [edge_index[0].astype(jnp.int32),
         jnp.zeros((E_PAD - E,), jnp.int32)]).reshape(NCHP, CHUNK)
    dst = jnp.concatenate(
        [edge_index[1].astype(jnp.int32),
         jnp.full((E_PAD - E,), N, jnp.int32)]).reshape(NCHP, CHUNK)

    eye4 = jnp.eye(4, dtype=jnp.float32)
    wAb = jnp.kron(eye4, W_emb[:, :HD])
    wBb = jnp.kron(eye4, W_emb[:, HD:])
    bA4 = jnp.tile(b_emb[:HD], 4).reshape(1, 4 * HD)
    bB4 = jnp.tile(b_emb[HD:], 4).reshape(1, 4 * HD)
    rsel = jnp.kron(eye4, jnp.ones((1, HD), jnp.float32))
    ssel = jnp.kron(jnp.ones((4, 1), jnp.float32),
                    jnp.eye(HD, dtype=jnp.float32))
    wfA = ssel @ W_fc[:HD]
    wfB = ssel @ W_fc[HD:]

    xp = jnp.pad(x.reshape(NP4, 4 * FEA), ((0, NPAD4 - NP4), (0, 0)))
    hA, hB = _emb(xp, wAb, wBb, bA4, bB4)

    cnt4 = None
    sA = sB = None
    for li, (Wl, bl, Wr) in enumerate(((Wl0, bl0, Wr0), (Wl1, bl1, Wr1),
                                       (Wl2, bl2, Wr2))):
        hAl = hA.reshape(NPAD, HD).astype(jnp.bfloat16)
        hBl = hB.reshape(NPAD, HD).astype(jnp.bfloat16)
        if li == 0:
            aggA, aggB, cnt = _SC_AGG_CNT(hAl, hBl, src, dst)
            cnt4 = cnt.reshape(NPAD4, 4)
        else:
            aggA, aggB = _SC_AGG(hAl, hBl, src, dst)
        args = (aggA.reshape(NPAD4, 4 * HD), aggB.reshape(NPAD4, 4 * HD),
                cnt4, rsel, hA, hB,
                _blockdiag(Wl[:HD, :HD]), _blockdiag(Wl[:HD, HD:]),
                _blockdiag(Wl[HD:, :HD]), _blockdiag(Wl[HD:, HD:]),
                _blockdiag(Wr[:HD, :HD]), _blockdiag(Wr[:HD, HD:]),
                _blockdiag(Wr[HD:, :HD]), _blockdiag(Wr[HD:, HD:]),
                jnp.tile(bl[:HD], 4).reshape(1, 4 * HD),
                jnp.tile(bl[HD:], 4).reshape(1, 4 * HD))
        if li < 2:
            hA, hB = _UPDATE(*args)
        else:
            hA, hB, sA, sB = _UPDATE_SUM(*args)
    out = _final(sA, sB, wfA, wfB, b_fc.reshape(1, H_DIM),
                 W_out, b_out.reshape(1, 1))
    return out.reshape(1)


# speedup vs baseline: 1.3672x; 1.0897x over previous
import jax
import jax.numpy as jnp
from jax import lax
from jax.experimental import pallas as pl
from jax.experimental.pallas import tpu as pltpu
from jax.experimental.pallas import tpu_sc as plsc

N = 50000
E = 800000
FEA = 128
HD = 32
H_DIM = 128
NS = 16
NPAD = 51200
STRIPE = NPAD // NS
CHUNK = 128
CPS = 396
IB = 12
NBLK = CPS // IB
NCHP = CPS * NS
E_PAD = NCHP * CHUNK
BLK = 400
GRID = N // BLK


def _sc_agg_build(with_cnt):
    mesh = plsc.VectorSubcoreMesh(core_axis_name="c", subcore_axis_name="s")
    outs = [jax.ShapeDtypeStruct((NPAD, HD), jnp.bfloat16),
            jax.ShapeDtypeStruct((NPAD, HD), jnp.bfloat16)]
    scratch = [
        pltpu.VMEM((3, IB, CHUNK), jnp.int32),
        pltpu.VMEM((3, IB, CHUNK), jnp.int32),
        pltpu.VMEM((CHUNK, HD), jnp.bfloat16),
        pltpu.VMEM((CHUNK, HD), jnp.bfloat16),
        pltpu.VMEM((CHUNK, HD), jnp.bfloat16),
        pltpu.VMEM((CHUNK, HD), jnp.bfloat16),
        pltpu.VMEM_SHARED((NPAD, HD), jnp.bfloat16),
        pltpu.SemaphoreType.DMA,
        pltpu.SemaphoreType.DMA,
        pltpu.SemaphoreType.DMA,
        pltpu.SemaphoreType.DMA,
        pltpu.SemaphoreType.DMA,
    ]
    if with_cnt:
        outs.append(jax.ShapeDtypeStruct((NPAD,), jnp.float32))
        scratch += [
            pltpu.VMEM((CHUNK,), jnp.float32),
            pltpu.VMEM((CHUNK,), jnp.float32),
            pltpu.VMEM_SHARED((NPAD,), jnp.float32),
        ]

    def body(hA, hB, src_h, dst_h, *rest):
        if with_cnt:
            (outA, outB, outC, src3, dst3, rows0, rows1, rows2, rows3,
             agg_sh, sem0, sem1, sem2, sem3, isem, cbuf, obuf,
             cnt_sh) = rest
        else:
            (outA, outB, src3, dst3, rows0, rows1, rows2, rows3, agg_sh,
             sem0, sem1, sem2, sem3, isem) = rest
        rows = (rows0, rows1, rows2, rows3)
        gsem = (sem0, sem1, sem2, sem3)
        cid = lax.axis_index("c")
        sid = lax.axis_index("s")
        z16 = jnp.zeros((16,), jnp.float32)
        zb32 = jnp.zeros((32,), jnp.bfloat16)

        def fill_row(i, c):
            rows0[i, pl.ds(0, 32)] = zb32
            return c
        lax.fori_loop(0, CHUNK, fill_row, 0)

        def zero_stripe(k, c):
            pltpu.sync_copy(rows0, agg_sh.at[pl.ds(sid * STRIPE + k * CHUNK, CHUNK)])
            return c
        lax.fori_loop(0, STRIPE // CHUNK, zero_stripe, 0)

        if with_cnt:
            o16 = jnp.ones((16,), jnp.float32)

            def fill_c(k, c):
                cbuf[pl.ds(k * 16, 16)] = z16
                obuf[pl.ds(k * 16, 16)] = o16
                return c
            lax.fori_loop(0, CHUNK // 16, fill_c, 0)

            @pl.when(cid == 0)
            def _():
                def zc(k, c):
                    pltpu.sync_copy(cbuf, cnt_sh.at[pl.ds(sid * STRIPE + k * CHUNK, CHUNK)])
                    return c
                lax.fori_loop(0, STRIPE // CHUNK, zc, 0)

        plsc.subcore_barrier()

        def main_loop(tab, do_cnt):
            def block(sl, nsl, noff, valid_next):
                for k in range(IB):
                    q = k % 4
                    pltpu.make_async_copy(tab.at[src3.at[sl, k]],
                                          rows[q], gsem[q]).wait()
                    nq = (k + 3) % 4
                    if k == IB - 4:
                        def wait_idx():
                            pltpu.make_async_copy(
                                src_h.at[pl.ds(noff, IB)], src3.at[nsl],
                                isem).wait()
                            pltpu.make_async_copy(
                                dst_h.at[pl.ds(noff, IB)], dst3.at[nsl],
                                isem).wait()
                        if valid_next is None:
                            wait_idx()
                        else:
                            pl.when(valid_next)(wait_idx)
                    if k < IB - 3:
                        pltpu.async_copy(tab.at[src3.at[sl, k + 3]],
                                         rows[nq], gsem[nq])
                    else:
                        def cross():
                            pltpu.async_copy(
                                tab.at[src3.at[nsl, k + 3 - IB]],
                                rows[nq], gsem[nq])
                        if valid_next is None:
                            cross()
                        else:
                            pl.when(valid_next)(cross)
                    pltpu.sync_copy(rows[q], agg_sh.at[dst3.at[sl, k]],
                                    add=True)
                    if do_cnt:
                        pltpu.sync_copy(obuf, cnt_sh.at[dst3.at[sl, k]],
                                        add=True)

            c0 = sid * CPS
            pltpu.sync_copy(src_h.at[pl.ds(c0, IB)], src3.at[0])
            pltpu.sync_copy(dst_h.at[pl.ds(c0, IB)], dst3.at[0])
            pltpu.async_copy(tab.at[src3.at[0, 0]], rows[0], gsem[0])
            pltpu.async_copy(tab.at[src3.at[0, 1]], rows[1], gsem[1])
            pltpu.async_copy(tab.at[src3.at[0, 2]], rows[2], gsem[2])

            def trpl(T, c):
                base = c0 + 3 * T * IB
                pltpu.async_copy(src_h.at[pl.ds(base + IB, IB)],
                                 src3.at[1], isem)
                pltpu.async_copy(dst_h.at[pl.ds(base + IB, IB)],
                                 dst3.at[1], isem)
                block(0, 1, base + IB, None)
                pltpu.async_copy(src_h.at[pl.ds(base + 2 * IB, IB)],
                                 src3.at[2], isem)
                pltpu.async_copy(dst_h.at[pl.ds(base + 2 * IB, IB)],
                                 dst3.at[2], isem)
                block(1, 2, base + 2 * IB, None)
                nxt = T + 1 < NBLK // 3

                @pl.when(nxt)
                def _():
                    pltpu.async_copy(src_h.at[pl.ds(base + 3 * IB, IB)],
                                     src3.at[0], isem)
                    pltpu.async_copy(dst_h.at[pl.ds(base + 3 * IB, IB)],
                                     dst3.at[0], isem)
                block(2, 0, base + 3 * IB, nxt)
                return c
            lax.fori_loop(0, NBLK // 3, trpl, 0)

        @pl.when(cid == 0)
        def _():
            main_loop(hA, with_cnt)

        @pl.when(cid == 1)
        def _():
            main_loop(hB, False)

        plsc.subcore_barrier()

        @pl.when(cid == 0)
        def _():
            pltpu.sync_copy(agg_sh.at[pl.ds(sid * STRIPE, STRIPE)],
                            outA.at[pl.ds(sid * STRIPE, STRIPE)])
            if with_cnt:
                pltpu.sync_copy(cnt_sh.at[pl.ds(sid * STRIPE, STRIPE)],
                                outC.at[pl.ds(sid * STRIPE, STRIPE)])

        @pl.when(cid == 1)
        def _():
            pltpu.sync_copy(agg_sh.at[pl.ds(sid * STRIPE, STRIPE)],
                            outB.at[pl.ds(sid * STRIPE, STRIPE)])

    return pl.kernel(body, out_type=tuple(outs), mesh=mesh,
                     scratch_types=tuple(scratch),
                     compiler_params=pltpu.CompilerParams(
                         use_tc_tiling_on_sc=False))


_SC_AGG_CNT = _sc_agg_build(True)
_SC_AGG = _sc_agg_build(False)


NP4 = N // 4
NPAD4 = NPAD // 4
BLK4 = 512
GRID4 = NPAD4 // BLK4


def _emb(xp, wAb, wBb, bA4, bB4):
    def body(x_r, wA_r, wB_r, bA_r, bB_r, oA_r, oB_r):
        xb = jnp.nan_to_num(x_r[...])
        oA_r[...] = jnp.dot(xb, wA_r[...],
                            preferred_element_type=jnp.float32, precision=lax.Precision.HIGHEST) + bA_r[...]
        oB_r[...] = jnp.dot(xb, wB_r[...],
                            preferred_element_type=jnp.float32, precision=lax.Precision.HIGHEST) + bB_r[...]

    blk = lambda i: (i, 0)
    one = lambda i: (0, 0)
    return pl.pallas_call(
        body,
        grid=(GRID4,),
        in_specs=[pl.BlockSpec((BLK4, 4 * FEA), blk),
                  pl.BlockSpec((4 * FEA, 4 * HD), one),
                  pl.BlockSpec((4 * FEA, 4 * HD), one),
                  pl.BlockSpec((1, 4 * HD), one),
                  pl.BlockSpec((1, 4 * HD), one)],
        out_specs=[pl.BlockSpec((BLK4, 4 * HD), blk)] * 2,
        out_shape=[jax.ShapeDtypeStruct((NPAD4, 4 * HD), jnp.float32)] * 2,
    )(xp, wAb, wBb, bA4, bB4)


def _update_build(with_sum):
    def body(aggA, aggB, cnt4, rsel, hA, hB, w00, w01, w10, w11,
             r00, r01, r10, r11, bA, bB, oA, oB, *s):
        i = pl.program_id(0)
        f32 = jnp.float32
        inv = jnp.dot(1.0 / jnp.maximum(cnt4[...], 1.0), rsel[...],
                      preferred_element_type=f32, precision=lax.Precision.HIGHEST)
        mA = aggA[...].astype(f32) * inv
        mB = aggB[...].astype(f32) * inv
        ha = hA[...]
        hb = hB[...]
        nA = (jnp.dot(mA, w00[...], preferred_element_type=f32, precision=lax.Precision.HIGHEST)
              + jnp.dot(mB, w10[...], preferred_element_type=f32, precision=lax.Precision.HIGHEST)
              + jnp.dot(ha, r00[...], preferred_element_type=f32, precision=lax.Precision.HIGHEST)
              + jnp.dot(hb, r10[...], preferred_element_type=f32, precision=lax.Precision.HIGHEST)
              + bA[...])
        nB = (jnp.dot(mA, w01[...], preferred_element_type=f32, precision=lax.Precision.HIGHEST)
              + jnp.dot(mB, w11[...], preferred_element_type=f32, precision=lax.Precision.HIGHEST)
              + jnp.dot(ha, r01[...], preferred_element_type=f32, precision=lax.Precision.HIGHEST)
              + jnp.dot(hb, r11[...], preferred_element_type=f32, precision=lax.Precision.HIGHEST)
              + bB[...])
        oA[...] = nA
        oB[...] = nB
        if with_sum:
            sA, sB = s

            @pl.when(i == 0)
            def _():
                sA[...] = jnp.zeros_like(sA)
                sB[...] = jnp.zeros_like(sB)

            live = (lax.broadcasted_iota(jnp.int32, (BLK4, 1), 0)
                    + i * BLK4) < NP4
            sA[...] += jnp.sum(jnp.where(live, nA, 0.0), axis=0,
                               keepdims=True)
            sB[...] += jnp.sum(jnp.where(live, nB, 0.0), axis=0,
                               keepdims=True)

    blk = lambda i: (i, 0)
    one = lambda i: (0, 0)
    in_specs = ([pl.BlockSpec((BLK4, 4 * HD), blk)] * 2
                + [pl.BlockSpec((BLK4, 4), blk),
                   pl.BlockSpec((4, 4 * HD), one)]
                + [pl.BlockSpec((BLK4, 4 * HD), blk)] * 2
                + [pl.BlockSpec((4 * HD, 4 * HD), one)] * 8
                + [pl.BlockSpec((1, 4 * HD), one)] * 2)
    out_specs = [pl.BlockSpec((BLK4, 4 * HD), blk)] * 2
    out_shape = [jax.ShapeDtypeStruct((NPAD4, 4 * HD), jnp.float32)] * 2
    if with_sum:
        out_specs += [pl.BlockSpec((1, 4 * HD), one)] * 2
        out_shape += [jax.ShapeDtypeStruct((1, 4 * HD), jnp.float32)] * 2

    return pl.pallas_call(body, grid=(GRID4,), in_specs=in_specs,
                          out_specs=out_specs, out_shape=out_shape)


_UPDATE = _update_build(False)
_UPDATE_SUM = _update_build(True)


def _final(sA, sB, wfA, wfB, bfc, wout, bout):
    def body(sA_r, sB_r, wfA_r, wfB_r, bfc_r, wo_r, bo_r, o_r):
        f32 = jnp.float32
        scale = 1.0 / N
        g = (jnp.dot(sA_r[...] * scale, wfA_r[...], preferred_element_type=f32, precision=lax.Precision.HIGHEST)
             + jnp.dot(sB_r[...] * scale, wfB_r[...], preferred_element_type=f32, precision=lax.Precision.HIGHEST)
             + bfc_r[...])
        sp = jnp.maximum(g, 0.0) + jnp.log(1.0 + jnp.exp(-jnp.abs(g)))
        o_r[...] = jnp.dot(sp, wo_r[...], preferred_element_type=f32, precision=lax.Precision.HIGHEST) + bo_r[...]

    return pl.pallas_call(
        body, out_shape=jax.ShapeDtypeStruct((1, 1), jnp.float32),
    )(sA, sB, wfA, wfB, bfc, wout, bout)


def _blockdiag(w):
    return jnp.kron(jnp.eye(4, dtype=w.dtype), w)


def kernel(x, edge_index, W_emb, b_emb, Wl0, bl0, Wr0, Wl1, bl1, Wr1,
           Wl2, bl2, Wr2, W_fc, b_fc, W_out, b_out):
    src = jnp.concatenate(
        [edge_index[0].astype(jnp.int32),
         jnp.zeros((E_PAD - E,), jnp.int32)]).reshape(NCHP, CHUNK)
    dst = jnp.concatenate(
        [edge_index[1].astype(jnp.int32),
         jnp.full((E_PAD - E,), N, jnp.int32)]).reshape(NCHP, CHUNK)

    eye4 = jnp.eye(4, dtype=jnp.float32)
    wAb = jnp.kron(eye4, W_emb[:, :HD])
    wBb = jnp.kron(eye4, W_emb[:, HD:])
    bA4 = jnp.tile(b_emb[:HD], 4).reshape(1, 4 * HD)
    bB4 = jnp.tile(b_emb[HD:], 4).reshape(1, 4 * HD)
    rsel = jnp.kron(eye4, jnp.ones((1, HD), jnp.float32))
    ssel = jnp.kron(jnp.ones((4, 1), jnp.float32),
                    jnp.eye(HD, dtype=jnp.float32))
    wfA = ssel @ W_fc[:HD]
    wfB = ssel @ W_fc[HD:]

    xp = jnp.pad(x.reshape(NP4, 4 * FEA), ((0, NPAD4 - NP4), (0, 0)))
    hA, hB = _emb(xp, wAb, wBb, bA4, bB4)

    cnt4 = None
    sA = sB = None
    for li, (Wl, bl, Wr) in enumerate(((Wl0, bl0, Wr0), (Wl1, bl1, Wr1),
                                       (Wl2, bl2, Wr2))):
        hAl = hA.reshape(NPAD, HD).astype(jnp.bfloat16)
        hBl = hB.reshape(NPAD, HD).astype(jnp.bfloat16)
        if li == 0:
            aggA, aggB, cnt = _SC_AGG_CNT(hAl, hBl, src, dst)
            cnt4 = cnt.reshape(NPAD4, 4)
        else:
            aggA, aggB = _SC_AGG(hAl, hBl, src, dst)
        args = (aggA.reshape(NPAD4, 4 * HD), aggB.reshape(NPAD4, 4 * HD),
                cnt4, rsel, hA, hB,
                _blockdiag(Wl[:HD, :HD]), _blockdiag(Wl[:HD, HD:]),
                _blockdiag(Wl[HD:, :HD]), _blockdiag(Wl[HD:, HD:]),
                _blockdiag(Wr[:HD, :HD]), _blockdiag(Wr[:HD, HD:]),
                _blockdiag(Wr[HD:, :HD]), _blockdiag(Wr[HD:, HD:]),
                jnp.tile(bl[:HD], 4).reshape(1, 4 * HD),
                jnp.tile(bl[HD:], 4).reshape(1, 4 * HD))
        if li < 2:
            hA, hB = _UPDATE(*args)
        else:
            hA, hB, sA, sB = _UPDATE_SUM(*args)
    out = _final(sA, sB, wfA, wfB, b_fc.reshape(1, H_DIM),
                 W_out, b_out.reshape(1, 1))
    return out.reshape(1)
